# baseline (speedup 1.0000x reference)
import functools

import jax
import jax.numpy as jnp
from jax import lax
from jax.experimental import pallas as pl
from jax.experimental.pallas import tpu as pltpu
from jax.experimental.pallas import tpu_sc as plsc

N = 10000
E = 320000
HID = 128
STAT = 16
EFEAT = 16
NUM_HEADS = 8
HEAD_SIZE = HID // NUM_HEADS
AUG = HID + STAT
ROW = HID + 2 * NUM_HEADS
MLP_WIDTH = 2 * HID

NC = 2
NS = 16
NW = NC * NS
EPW = E // NW
K = 80
NB = EPW // K
NPAD = 10240
ZCH = NPAD // NS
DCH = N // NS

_SC_MESH = dict(core_axis_name="c", subcore_axis_name="s",
                num_cores=NC, num_subcores=NS)


def _wid():
    return lax.axis_index("s") * NC + lax.axis_index("c")


def _gather_body(haug, srcf, dstf, gs_out, gd_out, idx_s, idx_d, bufs, bufd,
                 sem0, sem1):
    base0 = _wid() * EPW

    def body(j, carry):
        base = base0 + j * K
        pltpu.sync_copy(srcf.at[pl.ds(base, K)], idx_s)
        pltpu.sync_copy(dstf.at[pl.ds(base, K)], idx_d)
        cp0 = pltpu.async_copy(haug.at[idx_s], bufs, sem0)
        cp1 = pltpu.async_copy(haug.at[idx_d], bufd, sem1)
        cp0.wait()
        cp1.wait()
        pltpu.sync_copy(bufs, gs_out.at[pl.ds(base, K)])
        pltpu.sync_copy(bufd, gd_out.at[pl.ds(base, K)])
        return carry

    lax.fori_loop(0, NB, body, 0)


def _scatter_body(h_tab, srcf, dstf, expsf, zrows, part_out, idxs, idxd, gbuf,
                  ebuf, msgbuf, acc, gsem):
    c = lax.axis_index("c")
    s = lax.axis_index("s")
    wid = s * NC + c
    pltpu.sync_copy(zrows, acc.at[pl.ds(s * ZCH, ZCH)])
    ebuf[pl.ds(K * 8, 16)] = jnp.zeros((16,), jnp.float32)
    plsc.subcore_barrier()

    def body(j, carry):
        base = wid * EPW + j * K
        pltpu.sync_copy(srcf.at[pl.ds(base, K)], idxs)
        pltpu.sync_copy(dstf.at[pl.ds(base, K)], idxd)
        pltpu.async_copy(h_tab.at[idxs], gbuf, gsem).wait()
        pltpu.sync_copy(expsf.at[pl.ds(base * 8, K * 8)],
                        ebuf.at[pl.ds(0, K * 8)])

        def row(i, rc):
            erow = ebuf[pl.ds(8 * i, 16)]
            for v in range(NUM_HEADS):
                ev = jnp.full((16,), erow[v], dtype=jnp.float32)
                hv = gbuf[i, pl.ds(16 * v, 16)]
                msgbuf[i, pl.ds(16 * v, 16)] = hv * ev
            msgbuf[i, pl.ds(HID, 16)] = erow
            return rc

        lax.fori_loop(0, K, row, 0)
        pltpu.sync_copy(msgbuf, acc.at[idxd], add=True)
        return carry

    lax.fori_loop(0, NB, body, 0)
    plsc.subcore_barrier()
    pltpu.sync_copy(acc.at[pl.ds(s * DCH, DCH)],
                    part_out.at[c].at[pl.ds(s * DCH, DCH)])


def _recip_gather_body(recip, dstf, out, idxd, rbuf, sem):
    base0 = _wid() * EPW

    def body(j, carry):
        base = base0 + j * K
        pltpu.sync_copy(dstf.at[pl.ds(base, K)], idxd)
        pltpu.async_copy(recip.at[idxd], rbuf, sem).wait()
        pltpu.sync_copy(rbuf, out.at[pl.ds(base, K)])
        return carry

    lax.fori_loop(0, NB, body, 0)


def _mlp_body(gs_ref, gd_ref, ef_ref, w1s_ref, w1d_ref, w1e_ref, w2_ref,
              out_ref):
    pre = jnp.dot(gs_ref[...], w1s_ref[...], preferred_element_type=jnp.float32)
    pre = pre + jnp.dot(gd_ref[...], w1d_ref[...],
                        preferred_element_type=jnp.float32)
    pre = pre + jnp.dot(ef_ref[...], w1e_ref[...],
                        preferred_element_type=jnp.float32)
    z = jnp.maximum(pre, 0.0)
    raw = jnp.dot(z, w2_ref[...], preferred_element_type=jnp.float32)
    sc = jnp.maximum(raw, 0.01 * raw) * (1.0 / jnp.sqrt(jnp.float32(HEAD_SIZE)))
    out_ref[...] = jnp.exp(sc)


def _reduce_body(p0_ref, p1_ref, r_ref, wout_ref, proj_ref, recip_ref):
    tot = p0_ref[...] + p1_ref[...]
    u = tot[:, :HID]
    se = tot[:, HID:HID + NUM_HEADS]
    rec = 1.0 / (se + 1e-9)
    recip_ref[...] = rec
    rep = jnp.dot(rec, r_ref[...], preferred_element_type=jnp.float32)
    proj_ref[...] = jnp.dot(u * rep, wout_ref[...],
                            preferred_element_type=jnp.float32)


def _mul_body(a_ref, b_ref, o_ref):
    o_ref[...] = a_ref[...] * b_ref[...]


def kernel(h, x_s, edge_index, edge_features, W1, W2, W_out):
    f32 = jnp.float32
    bf16 = jnp.bfloat16
    haug = jnp.concatenate([h, x_s], axis=1).astype(bf16)
    src = edge_index[0]
    dst = edge_index[1]
    w1sT = jnp.concatenate([W1[:, :HID], W1[:, 2 * HID:2 * HID + STAT]],
                           axis=1).T.astype(bf16)
    w1dT = jnp.concatenate([W1[:, HID:2 * HID],
                            W1[:, 2 * HID + STAT:2 * HID + 2 * STAT]],
                           axis=1).T.astype(bf16)
    w1eT = W1[:, 2 * HID + 2 * STAT:].T
    w2T = W2.T
    woutT = W_out.T
    rbc = jnp.repeat(jnp.eye(NUM_HEADS, dtype=f32), HEAD_SIZE, axis=1)
    zrows = jnp.zeros((ZCH, ROW), f32)

    mesh = plsc.VectorSubcoreMesh(**_SC_MESH)

    gs, gd = pl.kernel(
        _gather_body,
        out_type=(jax.ShapeDtypeStruct((E, AUG), bf16),
                  jax.ShapeDtypeStruct((E, AUG), bf16)),
        mesh=mesh,
        compiler_params=pltpu.CompilerParams(use_tc_tiling_on_sc=False),
        scratch_types=[
            pltpu.VMEM((K,), jnp.int32),
            pltpu.VMEM((K,), jnp.int32),
            pltpu.VMEM((K, AUG), bf16),
            pltpu.VMEM((K, AUG), bf16),
            pltpu.SemaphoreType.DMA,
            pltpu.SemaphoreType.DMA,
        ],
    )(haug, src, dst)

    be = 2000
    exps = pl.pallas_call(
        _mlp_body,
        grid=(E // be,),
        in_specs=[
            pl.BlockSpec((be, AUG), lambda i: (i, 0)),
            pl.BlockSpec((be, AUG), lambda i: (i, 0)),
            pl.BlockSpec((be, EFEAT), lambda i: (i, 0)),
            pl.BlockSpec((AUG, MLP_WIDTH), lambda i: (0, 0)),
            pl.BlockSpec((AUG, MLP_WIDTH), lambda i: (0, 0)),
            pl.BlockSpec((EFEAT, MLP_WIDTH), lambda i: (0, 0)),
            pl.BlockSpec((MLP_WIDTH, NUM_HEADS), lambda i: (0, 0)),
        ],
        out_specs=pl.BlockSpec((be, NUM_HEADS), lambda i: (i, 0)),
        out_shape=jax.ShapeDtypeStruct((E, NUM_HEADS), f32),
    )(gs, gd, edge_features, w1sT, w1dT, w1eT, w2T)

    parts = pl.kernel(
        _scatter_body,
        out_type=jax.ShapeDtypeStruct((NC, N, ROW), f32),
        mesh=mesh,
        compiler_params=pltpu.CompilerParams(use_tc_tiling_on_sc=False),
        scratch_types=[
            pltpu.VMEM((K,), jnp.int32),
            pltpu.VMEM((K,), jnp.int32),
            pltpu.VMEM((K, HID), f32),
            pltpu.VMEM((K * 8 + 16,), f32),
            pltpu.VMEM((K, ROW), f32),
            pltpu.VMEM_SHARED((NPAD, ROW), f32),
            pltpu.SemaphoreType.DMA,
        ],
    )(h, src, dst, exps.reshape(-1), zrows)

    bn = 400
    proj, recip = pl.pallas_call(
        _reduce_body,
        grid=(N // bn,),
        in_specs=[
            pl.BlockSpec((bn, ROW), lambda i: (i, 0)),
            pl.BlockSpec((bn, ROW), lambda i: (i, 0)),
            pl.BlockSpec((NUM_HEADS, HID), lambda i: (0, 0)),
            pl.BlockSpec((HID, HID), lambda i: (0, 0)),
        ],
        out_specs=[
            pl.BlockSpec((bn, HID), lambda i: (i, 0)),
            pl.BlockSpec((bn, NUM_HEADS), lambda i: (i, 0)),
        ],
        out_shape=[
            jax.ShapeDtypeStruct((N, HID), f32),
            jax.ShapeDtypeStruct((N, NUM_HEADS), f32),
        ],
    )(parts[0], parts[1], rbc, woutT)

    recipg = pl.kernel(
        _recip_gather_body,
        out_type=jax.ShapeDtypeStruct((E, NUM_HEADS), f32),
        mesh=mesh,
        compiler_params=pltpu.CompilerParams(use_tc_tiling_on_sc=False),
        scratch_types=[
            pltpu.VMEM((K,), jnp.int32),
            pltpu.VMEM((K, NUM_HEADS), f32),
            pltpu.SemaphoreType.DMA,
        ],
    )(recip, dst)

    rows = E * NUM_HEADS // 128
    bw = 2000
    weights = pl.pallas_call(
        _mul_body,
        grid=(rows // bw,),
        in_specs=[
            pl.BlockSpec((bw, 128), lambda i: (i, 0)),
            pl.BlockSpec((bw, 128), lambda i: (i, 0)),
        ],
        out_specs=pl.BlockSpec((bw, 128), lambda i: (i, 0)),
        out_shape=jax.ShapeDtypeStruct((rows, 128), f32),
    )(exps.reshape(rows, 128), recipg.reshape(rows, 128))

    return (proj, weights.reshape(E, NUM_HEADS))


# speedup vs baseline: 1.2495x; 1.2495x over previous
import functools

import jax
import jax.numpy as jnp
from jax import lax
from jax.experimental import pallas as pl
from jax.experimental.pallas import tpu as pltpu
from jax.experimental.pallas import tpu_sc as plsc

N = 10000
E = 320000
HID = 128
STAT = 16
EFEAT = 16
NUM_HEADS = 8
HEAD_SIZE = HID // NUM_HEADS
AUG = HID + STAT
ROW = HID + 2 * NUM_HEADS
MLP_WIDTH = 2 * HID

NC = 2
NS = 16
NW = NC * NS
EPW = E // NW
K = 80
NB = EPW // K
NPAD = 10240
ZCH = NPAD // NS
DCH = N // NS

_SC_MESH = dict(core_axis_name="c", subcore_axis_name="s",
                num_cores=NC, num_subcores=NS)


def _wid():
    return lax.axis_index("s") * NC + lax.axis_index("c")


def _gather_body(haug, srcf, dstf, gs_out, gd_out, idx_s, idx_d, bufs, bufd,
                 sem0, sem1):
    base0 = _wid() * EPW

    def body(j, carry):
        base = base0 + j * K
        pltpu.sync_copy(srcf.at[pl.ds(base, K)], idx_s)
        pltpu.sync_copy(dstf.at[pl.ds(base, K)], idx_d)
        cp0 = pltpu.async_copy(haug.at[idx_s], bufs, sem0)
        cp1 = pltpu.async_copy(haug.at[idx_d], bufd, sem1)
        cp0.wait()
        cp1.wait()
        pltpu.sync_copy(bufs, gs_out.at[pl.ds(base, K)])
        pltpu.sync_copy(bufd, gd_out.at[pl.ds(base, K)])
        return carry

    lax.fori_loop(0, NB, body, 0)


def _scatter_body(gs, dstf, expsf, zrows, part_out, idxd, gbuf,
                  ebuf, msgbuf, acc):
    c = lax.axis_index("c")
    s = lax.axis_index("s")
    wid = s * NC + c
    pltpu.sync_copy(zrows, acc.at[pl.ds(s * ZCH, ZCH)])
    ebuf[pl.ds(K * 8, 16)] = jnp.zeros((16,), jnp.float32)
    plsc.subcore_barrier()

    def body(j, carry):
        base = wid * EPW + j * K
        pltpu.sync_copy(dstf.at[pl.ds(base, K)], idxd)
        pltpu.sync_copy(gs.at[pl.ds(base, K)], gbuf)
        pltpu.sync_copy(expsf.at[pl.ds(base * 8, K * 8)],
                        ebuf.at[pl.ds(0, K * 8)])

        def row(i, rc):
            erow = ebuf[pl.ds(8 * i, 16)]
            for v in range(NUM_HEADS):
                ev = jnp.full((16,), erow[v], dtype=jnp.float32)
                hv = gbuf[i, pl.ds(16 * v, 16)]
                msgbuf[i, pl.ds(16 * v, 16)] = hv * ev
            msgbuf[i, pl.ds(HID, 16)] = erow
            return rc

        lax.fori_loop(0, K, row, 0)
        pltpu.sync_copy(msgbuf, acc.at[idxd], add=True)
        return carry

    lax.fori_loop(0, NB, body, 0)
    plsc.subcore_barrier()
    pltpu.sync_copy(acc.at[pl.ds(s * DCH, DCH)],
                    part_out.at[c].at[pl.ds(s * DCH, DCH)])


def _recip_gather_body(recip, dstf, out, idxd, rbuf, sem):
    base0 = _wid() * EPW

    def body(j, carry):
        base = base0 + j * K
        pltpu.sync_copy(dstf.at[pl.ds(base, K)], idxd)
        pltpu.async_copy(recip.at[idxd], rbuf, sem).wait()
        pltpu.sync_copy(rbuf, out.at[pl.ds(base, K)])
        return carry

    lax.fori_loop(0, NB, body, 0)


def _mlp_body(gs_ref, gd_ref, ef_ref, w1s_ref, w1d_ref, w1e_ref, w2_ref,
              out_ref):
    bf16 = jnp.bfloat16
    pre = jnp.dot(gs_ref[...].astype(bf16), w1s_ref[...],
                  preferred_element_type=jnp.float32)
    pre = pre + jnp.dot(gd_ref[...].astype(bf16), w1d_ref[...],
                        preferred_element_type=jnp.float32)
    pre = pre + jnp.dot(ef_ref[...].astype(bf16), w1e_ref[...],
                        preferred_element_type=jnp.float32)
    z = jnp.maximum(pre, 0.0)
    raw = jnp.dot(z.astype(bf16), w2_ref[...],
                  preferred_element_type=jnp.float32)
    sc = jnp.maximum(raw, 0.01 * raw) * (1.0 / jnp.sqrt(jnp.float32(HEAD_SIZE)))
    out_ref[...] = jnp.exp(sc)


def _reduce_body(p0_ref, p1_ref, r_ref, wout_ref, proj_ref, recip_ref):
    tot = p0_ref[...] + p1_ref[...]
    u = tot[:, :HID]
    se = tot[:, HID:HID + NUM_HEADS]
    rec = 1.0 / (se + 1e-9)
    recip_ref[...] = rec
    rep = jnp.dot(rec, r_ref[...], preferred_element_type=jnp.float32)
    proj_ref[...] = jnp.dot(u * rep, wout_ref[...],
                            preferred_element_type=jnp.float32)


def _mul_body(a_ref, b_ref, o_ref):
    o_ref[...] = a_ref[...] * b_ref[...]


def kernel(h, x_s, edge_index, edge_features, W1, W2, W_out):
    f32 = jnp.float32
    bf16 = jnp.bfloat16
    haug = jnp.concatenate([h, x_s], axis=1)
    src = edge_index[0]
    dst = edge_index[1]
    w1sT = jnp.concatenate([W1[:, :HID], W1[:, 2 * HID:2 * HID + STAT]],
                           axis=1).T.astype(bf16)
    w1dT = jnp.concatenate([W1[:, HID:2 * HID],
                            W1[:, 2 * HID + STAT:2 * HID + 2 * STAT]],
                           axis=1).T.astype(bf16)
    w1eT = W1[:, 2 * HID + 2 * STAT:].T.astype(bf16)
    w2T = W2.T.astype(bf16)
    woutT = W_out.T
    rbc = jnp.repeat(jnp.eye(NUM_HEADS, dtype=f32), HEAD_SIZE, axis=1)
    zrows = jnp.zeros((ZCH, ROW), f32)

    mesh = plsc.VectorSubcoreMesh(**_SC_MESH)

    gs, gd = pl.kernel(
        _gather_body,
        out_type=(jax.ShapeDtypeStruct((E, AUG), f32),
                  jax.ShapeDtypeStruct((E, AUG), f32)),
        mesh=mesh,
        compiler_params=pltpu.CompilerParams(use_tc_tiling_on_sc=False),
        scratch_types=[
            pltpu.VMEM((K,), jnp.int32),
            pltpu.VMEM((K,), jnp.int32),
            pltpu.VMEM((K, AUG), f32),
            pltpu.VMEM((K, AUG), f32),
            pltpu.SemaphoreType.DMA,
            pltpu.SemaphoreType.DMA,
        ],
    )(haug, src, dst)

    be = 2000
    exps = pl.pallas_call(
        _mlp_body,
        grid=(E // be,),
        in_specs=[
            pl.BlockSpec((be, AUG), lambda i: (i, 0)),
            pl.BlockSpec((be, AUG), lambda i: (i, 0)),
            pl.BlockSpec((be, EFEAT), lambda i: (i, 0)),
            pl.BlockSpec((AUG, MLP_WIDTH), lambda i: (0, 0)),
            pl.BlockSpec((AUG, MLP_WIDTH), lambda i: (0, 0)),
            pl.BlockSpec((EFEAT, MLP_WIDTH), lambda i: (0, 0)),
            pl.BlockSpec((MLP_WIDTH, NUM_HEADS), lambda i: (0, 0)),
        ],
        out_specs=pl.BlockSpec((be, NUM_HEADS), lambda i: (i, 0)),
        out_shape=jax.ShapeDtypeStruct((E, NUM_HEADS), f32),
    )(gs, gd, edge_features, w1sT, w1dT, w1eT, w2T)

    parts = pl.kernel(
        _scatter_body,
        out_type=jax.ShapeDtypeStruct((NC, N, ROW), f32),
        mesh=mesh,
        compiler_params=pltpu.CompilerParams(use_tc_tiling_on_sc=False),
        scratch_types=[
            pltpu.VMEM((K,), jnp.int32),
            pltpu.VMEM((K, AUG), f32),
            pltpu.VMEM((K * 8 + 16,), f32),
            pltpu.VMEM((K, ROW), f32),
            pltpu.VMEM_SHARED((NPAD, ROW), f32),
        ],
    )(gs, dst, exps.reshape(-1), zrows)

    bn = 400
    proj, recip = pl.pallas_call(
        _reduce_body,
        grid=(N // bn,),
        in_specs=[
            pl.BlockSpec((bn, ROW), lambda i: (i, 0)),
            pl.BlockSpec((bn, ROW), lambda i: (i, 0)),
            pl.BlockSpec((NUM_HEADS, HID), lambda i: (0, 0)),
            pl.BlockSpec((HID, HID), lambda i: (0, 0)),
        ],
        out_specs=[
            pl.BlockSpec((bn, HID), lambda i: (i, 0)),
            pl.BlockSpec((bn, NUM_HEADS), lambda i: (i, 0)),
        ],
        out_shape=[
            jax.ShapeDtypeStruct((N, HID), f32),
            jax.ShapeDtypeStruct((N, NUM_HEADS), f32),
        ],
    )(parts[0], parts[1], rbc, woutT)

    recipg = pl.kernel(
        _recip_gather_body,
        out_type=jax.ShapeDtypeStruct((E, NUM_HEADS), f32),
        mesh=mesh,
        compiler_params=pltpu.CompilerParams(use_tc_tiling_on_sc=False),
        scratch_types=[
            pltpu.VMEM((K,), jnp.int32),
            pltpu.VMEM((K, NUM_HEADS), f32),
            pltpu.SemaphoreType.DMA,
        ],
    )(recip, dst)

    rows = E * NUM_HEADS // 128
    bw = 2000
    weights = pl.pallas_call(
        _mul_body,
        grid=(rows // bw,),
        in_specs=[
            pl.BlockSpec((bw, 128), lambda i: (i, 0)),
            pl.BlockSpec((bw, 128), lambda i: (i, 0)),
        ],
        out_specs=pl.BlockSpec((bw, 128), lambda i: (i, 0)),
        out_shape=jax.ShapeDtypeStruct((rows, 128), f32),
    )(exps.reshape(rows, 128), recipg.reshape(rows, 128))

    return (proj, weights.reshape(E, NUM_HEADS))


# speedup vs baseline: 1.2854x; 1.0287x over previous
import functools

import jax
import jax.numpy as jnp
from jax import lax
from jax.experimental import pallas as pl
from jax.experimental.pallas import tpu as pltpu
from jax.experimental.pallas import tpu_sc as plsc

N = 10000
E = 320000
HID = 128
STAT = 16
EFEAT = 16
NUM_HEADS = 8
HEAD_SIZE = HID // NUM_HEADS
ROW = HID + 2 * NUM_HEADS
MLP_WIDTH = 2 * HID

NC = 2
NS = 16
NW = NC * NS
NHALF = 2
EH = E // NHALF
EPW = EH // NW
K = 40
NB = EPW // K
KR = 80
NBR = (E // NW) // KR
NPAD = 10240
ZCH = NPAD // NS
DCH = N // NS

_SC_MESH = dict(core_axis_name="c", subcore_axis_name="s",
                num_cores=NC, num_subcores=NS)
_CP = dict(compiler_params=pltpu.CompilerParams(use_tc_tiling_on_sc=False))


def _wid():
    return lax.axis_index("s") * NC + lax.axis_index("c")


def _gather_body(off, h_tab, xs_tab, srcf, dstf, ghs_out, ghd_out, xp_out,
                 idx_s, idx_d, bufs, bufd, bufxs, bufxd, xpbuf,
                 sem0, sem1, sem2, sem3):
    base0 = _wid() * EPW

    def body(j, carry):
        base = base0 + j * K
        src_base = off + base
        pltpu.sync_copy(srcf.at[pl.ds(src_base, K)], idx_s)
        pltpu.sync_copy(dstf.at[pl.ds(src_base, K)], idx_d)
        cp0 = pltpu.async_copy(h_tab.at[idx_s], bufs, sem0)
        cp1 = pltpu.async_copy(h_tab.at[idx_d], bufd, sem1)
        cp2 = pltpu.async_copy(xs_tab.at[idx_s], bufxs, sem2)
        cp3 = pltpu.async_copy(xs_tab.at[idx_d], bufxd, sem3)
        cp0.wait()
        cp1.wait()
        cp2.wait()
        cp3.wait()
        pltpu.sync_copy(bufs, ghs_out.at[pl.ds(base, K)])
        pltpu.sync_copy(bufd, ghd_out.at[pl.ds(base, K)])
        def pack(i, pc):
            xpbuf[i, pl.ds(0, STAT)] = bufxs[i, pl.ds(0, STAT)]
            xpbuf[i, pl.ds(STAT, STAT)] = bufxd[i, pl.ds(0, STAT)]
            return pc

        lax.fori_loop(0, K, pack, 0)
        pltpu.sync_copy(xpbuf, xp_out.at[pl.ds(base, K)])
        return carry

    lax.fori_loop(0, NB, body, 0)


def _scatter_body(off, gs, dstf, expsf, zrows, part_out, idxd, gbuf,
                  ebuf, msgbuf, acc):
    c = lax.axis_index("c")
    s = lax.axis_index("s")
    wid = s * NC + c
    pltpu.sync_copy(zrows, acc.at[pl.ds(s * ZCH, ZCH)])
    ebuf[pl.ds(K * 8, 16)] = jnp.zeros((16,), jnp.float32)
    plsc.subcore_barrier()

    def body(j, carry):
        base = wid * EPW + j * K
        pltpu.sync_copy(dstf.at[pl.ds(off + base, K)], idxd)
        pltpu.sync_copy(gs.at[pl.ds(base, K)], gbuf)
        pltpu.sync_copy(expsf.at[pl.ds(base * 8, K * 8)],
                        ebuf.at[pl.ds(0, K * 8)])

        def row(i, rc):
            erow = ebuf[pl.ds(8 * i, 16)]
            for v in range(NUM_HEADS):
                ev = jnp.full((16,), erow[v], dtype=jnp.float32)
                hv = gbuf[i, pl.ds(16 * v, 16)]
                msgbuf[i, pl.ds(16 * v, 16)] = hv * ev
            msgbuf[i, pl.ds(HID, 16)] = erow
            return rc

        lax.fori_loop(0, K, row, 0)
        pltpu.sync_copy(msgbuf, acc.at[idxd], add=True)
        return carry

    lax.fori_loop(0, NB, body, 0)
    plsc.subcore_barrier()
    pltpu.sync_copy(acc.at[pl.ds(s * DCH, DCH)],
                    part_out.at[c].at[pl.ds(s * DCH, DCH)])


def _recip_gather_body(recip, dstf, out, idxd, rbuf, sem):
    base0 = _wid() * (E // NW)

    def body(j, carry):
        base = base0 + j * KR
        pltpu.sync_copy(dstf.at[pl.ds(base, KR)], idxd)
        pltpu.async_copy(recip.at[idxd], rbuf, sem).wait()
        pltpu.sync_copy(rbuf, out.at[pl.ds(base, KR)])
        return carry

    lax.fori_loop(0, NBR, body, 0)


def _mlp_body(gs_ref, gd_ref, xp_ref, ef_ref, w1s_ref, w1d_ref, w1x_ref,
              w1e_ref, w2_ref, out_ref):
    bf16 = jnp.bfloat16
    pre = jnp.dot(gs_ref[...].astype(bf16), w1s_ref[...],
                  preferred_element_type=jnp.float32)
    pre = pre + jnp.dot(gd_ref[...].astype(bf16), w1d_ref[...],
                        preferred_element_type=jnp.float32)
    pre = pre + jnp.dot(xp_ref[:, :2 * STAT].astype(bf16), w1x_ref[...],
                        preferred_element_type=jnp.float32)
    pre = pre + jnp.dot(ef_ref[...].astype(bf16), w1e_ref[...],
                        preferred_element_type=jnp.float32)
    z = jnp.maximum(pre, 0.0)
    raw = jnp.dot(z.astype(bf16), w2_ref[...],
                  preferred_element_type=jnp.float32)
    sc = jnp.maximum(raw, 0.01 * raw) * (1.0 / jnp.sqrt(jnp.float32(HEAD_SIZE)))
    out_ref[...] = jnp.exp(sc)


def _reduce_body(pa0_ref, pa1_ref, pb0_ref, pb1_ref, r_ref, wout_ref,
                 proj_ref, recip_ref):
    tot = (pa0_ref[...] + pa1_ref[...]) + (pb0_ref[...] + pb1_ref[...])
    u = tot[:, :HID]
    se = tot[:, HID:HID + NUM_HEADS]
    rec = 1.0 / (se + 1e-9)
    recip_ref[...] = rec
    rep = jnp.dot(rec, r_ref[...], preferred_element_type=jnp.float32)
    proj_ref[...] = jnp.dot(u * rep, wout_ref[...],
                            preferred_element_type=jnp.float32)


def _mul_body(a_ref, b_ref, o_ref):
    o_ref[...] = a_ref[...] * b_ref[...]


def kernel(h, x_s, edge_index, edge_features, W1, W2, W_out):
    f32 = jnp.float32
    bf16 = jnp.bfloat16
    src = edge_index[0]
    dst = edge_index[1]
    w1sT = W1[:, :HID].T.astype(bf16)
    w1dT = W1[:, HID:2 * HID].T.astype(bf16)
    w1xT = W1[:, 2 * HID:2 * HID + 2 * STAT].T.astype(bf16)
    w1eT = W1[:, 2 * HID + 2 * STAT:].T.astype(bf16)
    w2T = W2.T.astype(bf16)
    woutT = W_out.T
    rbc = jnp.repeat(jnp.eye(NUM_HEADS, dtype=f32), HEAD_SIZE, axis=1)
    zrows = jnp.zeros((ZCH, ROW), f32)

    mesh = plsc.VectorSubcoreMesh(**_SC_MESH)

    def gather_half(off):
        return pl.kernel(
            functools.partial(_gather_body, off),
            out_type=(jax.ShapeDtypeStruct((EH, HID), f32),
                      jax.ShapeDtypeStruct((EH, HID), f32),
                      jax.ShapeDtypeStruct((EH, HID), f32)),
            mesh=mesh, **_CP,
            scratch_types=[
                pltpu.VMEM((K,), jnp.int32),
                pltpu.VMEM((K,), jnp.int32),
                pltpu.VMEM((K, HID), f32),
                pltpu.VMEM((K, HID), f32),
                pltpu.VMEM((K, STAT), f32),
                pltpu.VMEM((K, STAT), f32),
                pltpu.VMEM((K, HID), f32),
                pltpu.SemaphoreType.DMA,
                pltpu.SemaphoreType.DMA,
                pltpu.SemaphoreType.DMA,
                pltpu.SemaphoreType.DMA,
            ],
        )(h, x_s, src, dst)

    be = 2000

    def mlp_half(gs, gd, xp, ef_half):
        return pl.pallas_call(
            _mlp_body,
            grid=(EH // be,),
            in_specs=[
                pl.BlockSpec((be, HID), lambda i: (i, 0)),
                pl.BlockSpec((be, HID), lambda i: (i, 0)),
                pl.BlockSpec((be, HID), lambda i: (i, 0)),
                pl.BlockSpec((be, EFEAT), lambda i: (i, 0)),
                pl.BlockSpec((HID, MLP_WIDTH), lambda i: (0, 0)),
                pl.BlockSpec((HID, MLP_WIDTH), lambda i: (0, 0)),
                pl.BlockSpec((2 * STAT, MLP_WIDTH), lambda i: (0, 0)),
                pl.BlockSpec((EFEAT, MLP_WIDTH), lambda i: (0, 0)),
                pl.BlockSpec((MLP_WIDTH, NUM_HEADS), lambda i: (0, 0)),
            ],
            out_specs=pl.BlockSpec((be, NUM_HEADS), lambda i: (i, 0)),
            out_shape=jax.ShapeDtypeStruct((EH, NUM_HEADS), f32),
        )(gs, gd, xp, ef_half, w1sT, w1dT, w1xT, w1eT, w2T)

    def scatter_half(off, gs, exps):
        return pl.kernel(
            functools.partial(_scatter_body, off),
            out_type=jax.ShapeDtypeStruct((NC, N, ROW), f32),
            mesh=mesh, **_CP,
            scratch_types=[
                pltpu.VMEM((K,), jnp.int32),
                pltpu.VMEM((K, HID), f32),
                pltpu.VMEM((K * 8 + 16,), f32),
                pltpu.VMEM((K, ROW), f32),
                pltpu.VMEM_SHARED((NPAD, ROW), f32),
            ],
        )(gs, dst, exps.reshape(-1), zrows)

    gsa, gda, xpa = gather_half(0)
    gsb, gdb, xpb = gather_half(EH)
    expsa = mlp_half(gsa, gda, xpa, edge_features[:EH])
    expsb = mlp_half(gsb, gdb, xpb, edge_features[EH:])
    parta = scatter_half(0, gsa, expsa)
    partb = scatter_half(EH, gsb, expsb)

    bn = 400
    proj, recip = pl.pallas_call(
        _reduce_body,
        grid=(N // bn,),
        in_specs=[
            pl.BlockSpec((bn, ROW), lambda i: (i, 0)),
            pl.BlockSpec((bn, ROW), lambda i: (i, 0)),
            pl.BlockSpec((bn, ROW), lambda i: (i, 0)),
            pl.BlockSpec((bn, ROW), lambda i: (i, 0)),
            pl.BlockSpec((NUM_HEADS, HID), lambda i: (0, 0)),
            pl.BlockSpec((HID, HID), lambda i: (0, 0)),
        ],
        out_specs=[
            pl.BlockSpec((bn, HID), lambda i: (i, 0)),
            pl.BlockSpec((bn, NUM_HEADS), lambda i: (i, 0)),
        ],
        out_shape=[
            jax.ShapeDtypeStruct((N, HID), f32),
            jax.ShapeDtypeStruct((N, NUM_HEADS), f32),
        ],
    )(parta[0], parta[1], partb[0], partb[1], rbc, woutT)

    recipg = pl.kernel(
        _recip_gather_body,
        out_type=jax.ShapeDtypeStruct((E, NUM_HEADS), f32),
        mesh=mesh, **_CP,
        scratch_types=[
            pltpu.VMEM((KR,), jnp.int32),
            pltpu.VMEM((KR, NUM_HEADS), f32),
            pltpu.SemaphoreType.DMA,
        ],
    )(recip, dst)

    exps = jnp.concatenate([expsa, expsb], axis=0)
    rows = E * NUM_HEADS // 128
    bw = 2000
    weights = pl.pallas_call(
        _mul_body,
        grid=(rows // bw,),
        in_specs=[
            pl.BlockSpec((bw, 128), lambda i: (i, 0)),
            pl.BlockSpec((bw, 128), lambda i: (i, 0)),
        ],
        out_specs=pl.BlockSpec((bw, 128), lambda i: (i, 0)),
        out_shape=jax.ShapeDtypeStruct((rows, 128), f32),
    )(exps.reshape(rows, 128), recipg.reshape(rows, 128))

    return (proj, weights.reshape(E, NUM_HEADS))


# speedup vs baseline: 1.5240x; 1.1857x over previous
import functools

import jax
import jax.numpy as jnp
from jax import lax
from jax.experimental import pallas as pl
from jax.experimental.pallas import tpu as pltpu
from jax.experimental.pallas import tpu_sc as plsc

N = 10000
E = 320000
HID = 128
STAT = 16
EFEAT = 16
NUM_HEADS = 8
HEAD_SIZE = HID // NUM_HEADS
ROW = HID + 2 * NUM_HEADS
MLP_WIDTH = 2 * HID

NC = 2
NS = 16
NW = NC * NS
EA = 192000
EB = E - EA
K = 80
KR = 80
NBR = (E // NW) // KR
NPAD = 10240
ZCH = NPAD // NS
DCH = N // NS

_SC_MESH = dict(core_axis_name="c", subcore_axis_name="s",
                num_cores=NC, num_subcores=NS)
_CP = dict(compiler_params=pltpu.CompilerParams(use_tc_tiling_on_sc=False))


def _wid():
    return lax.axis_index("s") * NC + lax.axis_index("c")


def _gather_body(off, epw, h_tab, xs_tab, srcf, dstf, ghs_out, ghd_out,
                 xp_out, idx_s, idx_d, bufs, bufd, bufxs, bufxd, xpbuf,
                 sem0, sem1, sem2, sem3):
    base0 = _wid() * epw

    def body(j, carry):
        base = base0 + j * K
        src_base = off + base
        pltpu.sync_copy(srcf.at[pl.ds(src_base, K)], idx_s)
        pltpu.sync_copy(dstf.at[pl.ds(src_base, K)], idx_d)
        cp0 = pltpu.async_copy(h_tab.at[idx_s], bufs, sem0)
        cp1 = pltpu.async_copy(h_tab.at[idx_d], bufd, sem1)
        cp2 = pltpu.async_copy(xs_tab.at[idx_s], bufxs, sem2)
        cp3 = pltpu.async_copy(xs_tab.at[idx_d], bufxd, sem3)
        cp0.wait()
        cp1.wait()
        cp2.wait()
        cp3.wait()
        pltpu.sync_copy(bufs, ghs_out.at[pl.ds(base, K)])
        pltpu.sync_copy(bufd, ghd_out.at[pl.ds(base, K)])
        def pack(i, pc):
            xpbuf[i, pl.ds(0, STAT)] = bufxs[i, pl.ds(0, STAT)]
            xpbuf[i, pl.ds(STAT, STAT)] = bufxd[i, pl.ds(0, STAT)]
            return pc

        lax.fori_loop(0, K, pack, 0)
        pltpu.sync_copy(xpbuf, xp_out.at[pl.ds(base, K)])
        return carry

    lax.fori_loop(0, epw // K, body, 0)


def _scatter_body(off, epw, gs, dstf, expsf, zrows, part_out, idxd, gbuf,
                  ebuf, msgbuf, acc):
    c = lax.axis_index("c")
    s = lax.axis_index("s")
    wid = s * NC + c
    pltpu.sync_copy(zrows, acc.at[pl.ds(s * ZCH, ZCH)])
    ebuf[pl.ds(K * 8, 16)] = jnp.zeros((16,), jnp.float32)
    plsc.subcore_barrier()

    def body(j, carry):
        base = wid * epw + j * K
        pltpu.sync_copy(dstf.at[pl.ds(off + base, K)], idxd)
        pltpu.sync_copy(gs.at[pl.ds(base, K)], gbuf)
        pltpu.sync_copy(expsf.at[pl.ds(base * 8, K * 8)],
                        ebuf.at[pl.ds(0, K * 8)])

        def row(i, rc):
            erow = ebuf[pl.ds(8 * i, 16)]
            for v in range(NUM_HEADS):
                ev = jnp.full((16,), erow[v], dtype=jnp.float32)
                hv = gbuf[i, pl.ds(16 * v, 16)]
                msgbuf[i, pl.ds(16 * v, 16)] = hv * ev
            msgbuf[i, pl.ds(HID, 16)] = erow
            return rc

        lax.fori_loop(0, K, row, 0)
        pltpu.sync_copy(msgbuf, acc.at[idxd], add=True)
        return carry

    lax.fori_loop(0, epw // K, body, 0)
    plsc.subcore_barrier()
    pltpu.sync_copy(acc.at[pl.ds(s * DCH, DCH)],
                    part_out.at[c].at[pl.ds(s * DCH, DCH)])


def _recip_gather_body(recip, dstf, out, idxd, rbuf, sem):
    base0 = _wid() * (E // NW)

    def body(j, carry):
        base = base0 + j * KR
        pltpu.sync_copy(dstf.at[pl.ds(base, KR)], idxd)
        pltpu.async_copy(recip.at[idxd], rbuf, sem).wait()
        pltpu.sync_copy(rbuf, out.at[pl.ds(base, KR)])
        return carry

    lax.fori_loop(0, NBR, body, 0)


def _mlp_body(gs_ref, gd_ref, xp_ref, ef_ref, w1s_ref, w1d_ref, w1x_ref,
              w1e_ref, w2_ref, out_ref):
    bf16 = jnp.bfloat16
    pre = jnp.dot(gs_ref[...].astype(bf16), w1s_ref[...],
                  preferred_element_type=jnp.float32)
    pre = pre + jnp.dot(gd_ref[...].astype(bf16), w1d_ref[...],
                        preferred_element_type=jnp.float32)
    pre = pre + jnp.dot(xp_ref[:, :2 * STAT].astype(bf16), w1x_ref[...],
                        preferred_element_type=jnp.float32)
    pre = pre + jnp.dot(ef_ref[...].astype(bf16), w1e_ref[...],
                        preferred_element_type=jnp.float32)
    z = jnp.maximum(pre, 0.0)
    raw = jnp.dot(z.astype(bf16), w2_ref[...],
                  preferred_element_type=jnp.float32)
    sc = jnp.maximum(raw, 0.01 * raw) * (1.0 / jnp.sqrt(jnp.float32(HEAD_SIZE)))
    out_ref[...] = jnp.exp(sc)


def _reduce_body(pa0_ref, pa1_ref, pb0_ref, pb1_ref, r_ref, wout_ref,
                 proj_ref, recip_ref):
    tot = (pa0_ref[...] + pa1_ref[...]) + (pb0_ref[...] + pb1_ref[...])
    u = tot[:, :HID]
    se = tot[:, HID:HID + NUM_HEADS]
    rec = 1.0 / (se + 1e-9)
    recip_ref[...] = rec
    rep = jnp.dot(rec, r_ref[...], preferred_element_type=jnp.float32)
    proj_ref[...] = jnp.dot(u * rep, wout_ref[...],
                            preferred_element_type=jnp.float32)


def _mul_body(a_ref, b_ref, o_ref):
    o_ref[...] = a_ref[...] * b_ref[...]


def kernel(h, x_s, edge_index, edge_features, W1, W2, W_out):
    f32 = jnp.float32
    bf16 = jnp.bfloat16
    src = edge_index[0]
    dst = edge_index[1]
    w1sT = W1[:, :HID].T.astype(bf16)
    w1dT = W1[:, HID:2 * HID].T.astype(bf16)
    w1xT = W1[:, 2 * HID:2 * HID + 2 * STAT].T.astype(bf16)
    w1eT = W1[:, 2 * HID + 2 * STAT:].T.astype(bf16)
    w2T = W2.T.astype(bf16)
    woutT = W_out.T
    rbc = jnp.repeat(jnp.eye(NUM_HEADS, dtype=f32), HEAD_SIZE, axis=1)
    zrows = jnp.zeros((ZCH, ROW), f32)

    mesh = plsc.VectorSubcoreMesh(**_SC_MESH)

    def gather_half(off, eh):
        return pl.kernel(
            functools.partial(_gather_body, off, eh // NW),
            out_type=(jax.ShapeDtypeStruct((eh, HID), f32),
                      jax.ShapeDtypeStruct((eh, HID), f32),
                      jax.ShapeDtypeStruct((eh, HID), f32)),
            mesh=mesh, **_CP,
            scratch_types=[
                pltpu.VMEM((K,), jnp.int32),
                pltpu.VMEM((K,), jnp.int32),
                pltpu.VMEM((K, HID), f32),
                pltpu.VMEM((K, HID), f32),
                pltpu.VMEM((K, STAT), f32),
                pltpu.VMEM((K, STAT), f32),
                pltpu.VMEM((K, HID), f32),
                pltpu.SemaphoreType.DMA,
                pltpu.SemaphoreType.DMA,
                pltpu.SemaphoreType.DMA,
                pltpu.SemaphoreType.DMA,
            ],
        )(h, x_s, src, dst)

    be = 2000

    def mlp_half(gs, gd, xp, ef_half):
        eh = gs.shape[0]
        return pl.pallas_call(
            _mlp_body,
            grid=(eh // be,),
            in_specs=[
                pl.BlockSpec((be, HID), lambda i: (i, 0)),
                pl.BlockSpec((be, HID), lambda i: (i, 0)),
                pl.BlockSpec((be, HID), lambda i: (i, 0)),
                pl.BlockSpec((be, EFEAT), lambda i: (i, 0)),
                pl.BlockSpec((HID, MLP_WIDTH), lambda i: (0, 0)),
                pl.BlockSpec((HID, MLP_WIDTH), lambda i: (0, 0)),
                pl.BlockSpec((2 * STAT, MLP_WIDTH), lambda i: (0, 0)),
                pl.BlockSpec((EFEAT, MLP_WIDTH), lambda i: (0, 0)),
                pl.BlockSpec((MLP_WIDTH, NUM_HEADS), lambda i: (0, 0)),
            ],
            out_specs=pl.BlockSpec((be, NUM_HEADS), lambda i: (i, 0)),
            out_shape=jax.ShapeDtypeStruct((eh, NUM_HEADS), f32),
        )(gs, gd, xp, ef_half, w1sT, w1dT, w1xT, w1eT, w2T)

    def scatter_half(off, gs, exps):
        return pl.kernel(
            functools.partial(_scatter_body, off, gs.shape[0] // NW),
            out_type=jax.ShapeDtypeStruct((NC, N, ROW), f32),
            mesh=mesh, **_CP,
            scratch_types=[
                pltpu.VMEM((K,), jnp.int32),
                pltpu.VMEM((K, HID), f32),
                pltpu.VMEM((K * 8 + 16,), f32),
                pltpu.VMEM((K, ROW), f32),
                pltpu.VMEM_SHARED((NPAD, ROW), f32),
            ],
        )(gs, dst, exps.reshape(-1), zrows)

    gsa, gda, xpa = gather_half(0, EA)
    gsb, gdb, xpb = gather_half(EA, EB)
    expsa = mlp_half(gsa, gda, xpa, edge_features[:EA])
    expsb = mlp_half(gsb, gdb, xpb, edge_features[EA:])
    parta = scatter_half(0, gsa, expsa)
    partb = scatter_half(EA, gsb, expsb)

    bn = 400
    proj, recip = pl.pallas_call(
        _reduce_body,
        grid=(N // bn,),
        in_specs=[
            pl.BlockSpec((bn, ROW), lambda i: (i, 0)),
            pl.BlockSpec((bn, ROW), lambda i: (i, 0)),
            pl.BlockSpec((bn, ROW), lambda i: (i, 0)),
            pl.BlockSpec((bn, ROW), lambda i: (i, 0)),
            pl.BlockSpec((NUM_HEADS, HID), lambda i: (0, 0)),
            pl.BlockSpec((HID, HID), lambda i: (0, 0)),
        ],
        out_specs=[
            pl.BlockSpec((bn, HID), lambda i: (i, 0)),
            pl.BlockSpec((bn, NUM_HEADS), lambda i: (i, 0)),
        ],
        out_shape=[
            jax.ShapeDtypeStruct((N, HID), f32),
            jax.ShapeDtypeStruct((N, NUM_HEADS), f32),
        ],
    )(parta[0], parta[1], partb[0], partb[1], rbc, woutT)

    recipg = pl.kernel(
        _recip_gather_body,
        out_type=jax.ShapeDtypeStruct((E, NUM_HEADS), f32),
        mesh=mesh, **_CP,
        scratch_types=[
            pltpu.VMEM((KR,), jnp.int32),
            pltpu.VMEM((KR, NUM_HEADS), f32),
            pltpu.SemaphoreType.DMA,
        ],
    )(recip, dst)

    exps = jnp.concatenate([expsa, expsb], axis=0)
    rows = E * NUM_HEADS // 128
    bw = 2000
    weights = pl.pallas_call(
        _mul_body,
        grid=(rows // bw,),
        in_specs=[
            pl.BlockSpec((bw, 128), lambda i: (i, 0)),
            pl.BlockSpec((bw, 128), lambda i: (i, 0)),
        ],
        out_specs=pl.BlockSpec((bw, 128), lambda i: (i, 0)),
        out_shape=jax.ShapeDtypeStruct((rows, 128), f32),
    )(exps.reshape(rows, 128), recipg.reshape(rows, 128))

    return (proj, weights.reshape(E, NUM_HEADS))


# speedup vs baseline: 1.8644x; 1.2234x over previous
import functools

import jax
import jax.numpy as jnp
from jax import lax
from jax.experimental import pallas as pl
from jax.experimental.pallas import tpu as pltpu
from jax.experimental.pallas import tpu_sc as plsc

N = 10000
E = 320000
HID = 128
STAT = 16
EFEAT = 16
NUM_HEADS = 8
HEAD_SIZE = HID // NUM_HEADS
ROW = HID + 2 * NUM_HEADS
MLP_WIDTH = 2 * HID

NC = 2
NS = 16
NW = NC * NS
EA = 192000
EB = E - EA
K = 80
KR = 80
NBR = (E // NW) // KR
NPAD = N
ZCH = NPAD // NS
DCH = N // NS

_SC_MESH = dict(core_axis_name="c", subcore_axis_name="s",
                num_cores=NC, num_subcores=NS)
_CP = dict(compiler_params=pltpu.CompilerParams(use_tc_tiling_on_sc=False))


def _wid():
    return lax.axis_index("s") * NC + lax.axis_index("c")


def _gather_body(off, epw, h_tab, xs_tab, srcf, dstf, ghs_out, ghd_out,
                 xp_out, idx_s, idx_d, bufs, bufd, bufxs, bufxd, xpbuf,
                 semi0, semi1, semg0, semg1, semw0, semw1):
    base0 = _wid() * epw
    nb_blocks = epw // K
    semi = (semi0, semi1)
    semg = (semg0, semg1)
    semw = (semw0, semw1)

    def idx_refs(b):
        return idx_s.at[b], idx_d.at[b]

    def issue_idx(j, b):
        base = off + base0 + j * K
        ii_s, ii_d = idx_refs(b)
        pltpu.async_copy(srcf.at[pl.ds(base, K)], ii_s, semi[b])
        pltpu.async_copy(dstf.at[pl.ds(base, K)], ii_d, semi[b])

    def wait_idx(b):
        ii_s, ii_d = idx_refs(b)
        pltpu.make_async_copy(srcf.at[pl.ds(0, K)], ii_s, semi[b]).wait()
        pltpu.make_async_copy(dstf.at[pl.ds(0, K)], ii_d, semi[b]).wait()

    def issue_gather(b):
        ii_s, ii_d = idx_refs(b)
        pltpu.async_copy(h_tab.at[ii_s], bufs.at[b], semg[b])
        pltpu.async_copy(h_tab.at[ii_d], bufd.at[b], semg[b])
        pltpu.async_copy(xs_tab.at[ii_s], bufxs.at[b], semg[b])
        pltpu.async_copy(xs_tab.at[ii_d], bufxd.at[b], semg[b])

    def wait_gather(b):
        ii_s, ii_d = idx_refs(b)
        pltpu.make_async_copy(h_tab.at[ii_s], bufs.at[b], semg[b]).wait()
        pltpu.make_async_copy(h_tab.at[ii_d], bufd.at[b], semg[b]).wait()
        pltpu.make_async_copy(xs_tab.at[ii_s], bufxs.at[b], semg[b]).wait()
        pltpu.make_async_copy(xs_tab.at[ii_d], bufxd.at[b], semg[b]).wait()

    def issue_write(j, b):
        base = base0 + j * K
        pltpu.async_copy(bufs.at[b], ghs_out.at[pl.ds(base, K)], semw[b])
        pltpu.async_copy(bufd.at[b], ghd_out.at[pl.ds(base, K)], semw[b])
        pltpu.async_copy(xpbuf.at[b], xp_out.at[pl.ds(base, K)], semw[b])

    def wait_write(b):
        pltpu.make_async_copy(bufs.at[b], ghs_out.at[pl.ds(0, K)],
                              semw[b]).wait()
        pltpu.make_async_copy(bufd.at[b], ghd_out.at[pl.ds(0, K)],
                              semw[b]).wait()
        pltpu.make_async_copy(xpbuf.at[b], xp_out.at[pl.ds(0, K)],
                              semw[b]).wait()

    def pack(b):
        def row(i, pc):
            xpbuf[b, i, pl.ds(0, STAT)] = bufxs[b, i, pl.ds(0, STAT)]
            xpbuf[b, i, pl.ds(STAT, STAT)] = bufxd[b, i, pl.ds(0, STAT)]
            return pc

        lax.fori_loop(0, K, row, 0)

    issue_idx(0, 0)
    issue_idx(1, 1)
    wait_idx(0)
    issue_gather(0)

    def body(j, carry):
        def steady(b, nb):
            wait_gather(b)
            pack(b)

            @pl.when(j >= 1)
            def _():
                wait_write(nb)

            @pl.when(j + 1 < nb_blocks)
            def _():
                wait_idx(nb)
                issue_gather(nb)

            @pl.when(j + 2 < nb_blocks)
            def _():
                issue_idx(j + 2, b)

            issue_write(j, b)

        @pl.when(lax.rem(j, 2) == 0)
        def _():
            steady(0, 1)

        @pl.when(lax.rem(j, 2) == 1)
        def _():
            steady(1, 0)

        return carry

    lax.fori_loop(0, nb_blocks, body, 0)
    wait_write((nb_blocks - 1) % 2)


def _scatter_body(off, epw, gs, dstf, expsf, zrows, part_out, idxd, gbuf,
                  ebuf, msgbuf, acc, semL0, semL1, semS):
    c = lax.axis_index("c")
    s = lax.axis_index("s")
    wid = s * NC + c
    pltpu.sync_copy(zrows, acc.at[pl.ds(s * ZCH, ZCH)])
    ebuf[0, pl.ds(K * 8, 16)] = jnp.zeros((16,), jnp.float32)
    ebuf[1, pl.ds(K * 8, 16)] = jnp.zeros((16,), jnp.float32)
    plsc.subcore_barrier()

    nb_blocks = epw // K
    semL = (semL0, semL1)

    def issue_loads(j, b):
        base = wid * epw + j * K
        pltpu.async_copy(dstf.at[pl.ds(off + base, K)], idxd.at[b], semL[b])
        pltpu.async_copy(gs.at[pl.ds(base, K)], gbuf.at[b], semL[b])
        pltpu.async_copy(expsf.at[pl.ds(base * 8, K * 8)],
                         ebuf.at[b].at[pl.ds(0, K * 8)], semL[b])

    def wait_loads(b):
        pltpu.make_async_copy(dstf.at[pl.ds(0, K)], idxd.at[b],
                              semL[b]).wait()
        pltpu.make_async_copy(gs.at[pl.ds(0, K)], gbuf.at[b], semL[b]).wait()
        pltpu.make_async_copy(expsf.at[pl.ds(0, K * 8)],
                              ebuf.at[b].at[pl.ds(0, K * 8)], semL[b]).wait()

    def wait_scatter(b):
        pltpu.make_async_copy(msgbuf, acc.at[idxd.at[b]], semS).wait()

    issue_loads(0, 0)

    def body(j, carry):
        def steady(b, nb):
            wait_loads(b)

            @pl.when(j >= 1)
            def _():
                wait_scatter(nb)

            @pl.when(j + 1 < nb_blocks)
            def _():
                issue_loads(j + 1, nb)

            def row(i, rc):
                erow = ebuf[b, pl.ds(8 * i, 16)]
                for v in range(NUM_HEADS):
                    ev = jnp.full((16,), erow[v], dtype=jnp.float32)
                    hv = gbuf[b, i, pl.ds(16 * v, 16)]
                    msgbuf[i, pl.ds(16 * v, 16)] = hv * ev
                msgbuf[i, pl.ds(HID, 16)] = erow
                return rc

            lax.fori_loop(0, K, row, 0)
            pltpu.async_copy(msgbuf, acc.at[idxd.at[b]], semS, add=True)

        @pl.when(lax.rem(j, 2) == 0)
        def _():
            steady(0, 1)

        @pl.when(lax.rem(j, 2) == 1)
        def _():
            steady(1, 0)

        return carry

    lax.fori_loop(0, nb_blocks, body, 0)
    wait_scatter((nb_blocks - 1) % 2)
    plsc.subcore_barrier()
    pltpu.sync_copy(acc.at[pl.ds(s * DCH, DCH)],
                    part_out.at[c].at[pl.ds(s * DCH, DCH)])


def _recip_gather_body(recip, dstf, out, idxd, rbuf, sem):
    base0 = _wid() * (E // NW)

    def body(j, carry):
        base = base0 + j * KR
        pltpu.sync_copy(dstf.at[pl.ds(base, KR)], idxd)
        pltpu.async_copy(recip.at[idxd], rbuf, sem).wait()
        pltpu.sync_copy(rbuf, out.at[pl.ds(base, KR)])
        return carry

    lax.fori_loop(0, NBR, body, 0)


def _mlp_body(gs_ref, gd_ref, xp_ref, ef_ref, w1s_ref, w1d_ref, w1x_ref,
              w1e_ref, w2_ref, out_ref):
    bf16 = jnp.bfloat16
    pre = jnp.dot(gs_ref[...].astype(bf16), w1s_ref[...],
                  preferred_element_type=jnp.float32)
    pre = pre + jnp.dot(gd_ref[...].astype(bf16), w1d_ref[...],
                        preferred_element_type=jnp.float32)
    pre = pre + jnp.dot(xp_ref[:, :2 * STAT].astype(bf16), w1x_ref[...],
                        preferred_element_type=jnp.float32)
    pre = pre + jnp.dot(ef_ref[...].astype(bf16), w1e_ref[...],
                        preferred_element_type=jnp.float32)
    z = jnp.maximum(pre, 0.0)
    raw = jnp.dot(z.astype(bf16), w2_ref[...],
                  preferred_element_type=jnp.float32)
    sc = jnp.maximum(raw, 0.01 * raw) * (1.0 / jnp.sqrt(jnp.float32(HEAD_SIZE)))
    out_ref[...] = jnp.exp(sc)


def _reduce_body(pa0_ref, pa1_ref, pb0_ref, pb1_ref, r_ref, wout_ref,
                 proj_ref, recip_ref):
    tot = (pa0_ref[...] + pa1_ref[...]) + (pb0_ref[...] + pb1_ref[...])
    u = tot[:, :HID]
    se = tot[:, HID:HID + NUM_HEADS]
    rec = 1.0 / (se + 1e-9)
    recip_ref[...] = rec
    rep = jnp.dot(rec, r_ref[...], preferred_element_type=jnp.float32)
    proj_ref[...] = jnp.dot(u * rep, wout_ref[...],
                            preferred_element_type=jnp.float32)


def _mul_body(a_ref, b_ref, o_ref):
    o_ref[...] = a_ref[...] * b_ref[...]


def kernel(h, x_s, edge_index, edge_features, W1, W2, W_out):
    f32 = jnp.float32
    bf16 = jnp.bfloat16
    src = edge_index[0]
    dst = edge_index[1]
    w1sT = W1[:, :HID].T.astype(bf16)
    w1dT = W1[:, HID:2 * HID].T.astype(bf16)
    w1xT = W1[:, 2 * HID:2 * HID + 2 * STAT].T.astype(bf16)
    w1eT = W1[:, 2 * HID + 2 * STAT:].T.astype(bf16)
    w2T = W2.T.astype(bf16)
    woutT = W_out.T
    rbc = jnp.repeat(jnp.eye(NUM_HEADS, dtype=f32), HEAD_SIZE, axis=1)
    zrows = jnp.zeros((ZCH, ROW), f32)

    mesh = plsc.VectorSubcoreMesh(**_SC_MESH)

    def gather_half(off, eh):
        return pl.kernel(
            functools.partial(_gather_body, off, eh // NW),
            out_type=(jax.ShapeDtypeStruct((eh, HID), f32),
                      jax.ShapeDtypeStruct((eh, HID), f32),
                      jax.ShapeDtypeStruct((eh, HID), f32)),
            mesh=mesh, **_CP,
            scratch_types=[
                pltpu.VMEM((2, K), jnp.int32),
                pltpu.VMEM((2, K), jnp.int32),
                pltpu.VMEM((2, K, HID), f32),
                pltpu.VMEM((2, K, HID), f32),
                pltpu.VMEM((2, K, STAT), f32),
                pltpu.VMEM((2, K, STAT), f32),
                pltpu.VMEM((2, K, HID), f32),
                pltpu.SemaphoreType.DMA,
                pltpu.SemaphoreType.DMA,
                pltpu.SemaphoreType.DMA,
                pltpu.SemaphoreType.DMA,
                pltpu.SemaphoreType.DMA,
                pltpu.SemaphoreType.DMA,
            ],
        )(h, x_s, src, dst)

    be = 2000

    def mlp_half(gs, gd, xp, ef_half):
        eh = gs.shape[0]
        return pl.pallas_call(
            _mlp_body,
            grid=(eh // be,),
            in_specs=[
                pl.BlockSpec((be, HID), lambda i: (i, 0)),
                pl.BlockSpec((be, HID), lambda i: (i, 0)),
                pl.BlockSpec((be, HID), lambda i: (i, 0)),
                pl.BlockSpec((be, EFEAT), lambda i: (i, 0)),
                pl.BlockSpec((HID, MLP_WIDTH), lambda i: (0, 0)),
                pl.BlockSpec((HID, MLP_WIDTH), lambda i: (0, 0)),
                pl.BlockSpec((2 * STAT, MLP_WIDTH), lambda i: (0, 0)),
                pl.BlockSpec((EFEAT, MLP_WIDTH), lambda i: (0, 0)),
                pl.BlockSpec((MLP_WIDTH, NUM_HEADS), lambda i: (0, 0)),
            ],
            out_specs=pl.BlockSpec((be, NUM_HEADS), lambda i: (i, 0)),
            out_shape=jax.ShapeDtypeStruct((eh, NUM_HEADS), f32),
        )(gs, gd, xp, ef_half, w1sT, w1dT, w1xT, w1eT, w2T)

    def scatter_half(off, gs, exps):
        return pl.kernel(
            functools.partial(_scatter_body, off, gs.shape[0] // NW),
            out_type=jax.ShapeDtypeStruct((NC, N, ROW), f32),
            mesh=mesh, **_CP,
            scratch_types=[
                pltpu.VMEM((2, K), jnp.int32),
                pltpu.VMEM((2, K, HID), f32),
                pltpu.VMEM((2, K * 8 + 16), f32),
                pltpu.VMEM((K, ROW), f32),
                pltpu.VMEM_SHARED((NPAD, ROW), f32),
                pltpu.SemaphoreType.DMA,
                pltpu.SemaphoreType.DMA,
                pltpu.SemaphoreType.DMA,
            ],
        )(gs, dst, exps.reshape(-1), zrows)

    gsa, gda, xpa = gather_half(0, EA)
    gsb, gdb, xpb = gather_half(EA, EB)
    expsa = mlp_half(gsa, gda, xpa, edge_features[:EA])
    expsb = mlp_half(gsb, gdb, xpb, edge_features[EA:])
    parta = scatter_half(0, gsa, expsa)
    partb = scatter_half(EA, gsb, expsb)

    bn = 400
    proj, recip = pl.pallas_call(
        _reduce_body,
        grid=(N // bn,),
        in_specs=[
            pl.BlockSpec((bn, ROW), lambda i: (i, 0)),
            pl.BlockSpec((bn, ROW), lambda i: (i, 0)),
            pl.BlockSpec((bn, ROW), lambda i: (i, 0)),
            pl.BlockSpec((bn, ROW), lambda i: (i, 0)),
            pl.BlockSpec((NUM_HEADS, HID), lambda i: (0, 0)),
            pl.BlockSpec((HID, HID), lambda i: (0, 0)),
        ],
        out_specs=[
            pl.BlockSpec((bn, HID), lambda i: (i, 0)),
            pl.BlockSpec((bn, NUM_HEADS), lambda i: (i, 0)),
        ],
        out_shape=[
            jax.ShapeDtypeStruct((N, HID), f32),
            jax.ShapeDtypeStruct((N, NUM_HEADS), f32),
        ],
    )(parta[0], parta[1], partb[0], partb[1], rbc, woutT)

    recipg = pl.kernel(
        _recip_gather_body,
        out_type=jax.ShapeDtypeStruct((E, NUM_HEADS), f32),
        mesh=mesh, **_CP,
        scratch_types=[
            pltpu.VMEM((KR,), jnp.int32),
            pltpu.VMEM((KR, NUM_HEADS), f32),
            pltpu.SemaphoreType.DMA,
        ],
    )(recip, dst)

    exps = jnp.concatenate([expsa, expsb], axis=0)
    rows = E * NUM_HEADS // 128
    bw = 2000
    weights = pl.pallas_call(
        _mul_body,
        grid=(rows // bw,),
        in_specs=[
            pl.BlockSpec((bw, 128), lambda i: (i, 0)),
            pl.BlockSpec((bw, 128), lambda i: (i, 0)),
        ],
        out_specs=pl.BlockSpec((bw, 128), lambda i: (i, 0)),
        out_shape=jax.ShapeDtypeStruct((rows, 128), f32),
    )(exps.reshape(rows, 128), recipg.reshape(rows, 128))

    return (proj, weights.reshape(E, NUM_HEADS))


# speedup vs baseline: 1.8902x; 1.0138x over previous
import functools

import jax
import jax.numpy as jnp
from jax import lax
from jax.experimental import pallas as pl
from jax.experimental.pallas import tpu as pltpu
from jax.experimental.pallas import tpu_sc as plsc

N = 10000
E = 320000
HID = 128
STAT = 16
EFEAT = 16
NUM_HEADS = 8
HEAD_SIZE = HID // NUM_HEADS
ROW = HID + 2 * NUM_HEADS
MLP_WIDTH = 2 * HID

NC = 2
NS = 16
NW = NC * NS
EA = 192000
EB = E - EA
K = 80
KR = 80
NBR = (E // NW) // KR
NPAD = N
ZCH = NPAD // NS
DCH = N // NS

_SC_MESH = dict(core_axis_name="c", subcore_axis_name="s",
                num_cores=NC, num_subcores=NS)
_CP = dict(compiler_params=pltpu.CompilerParams(use_tc_tiling_on_sc=False))


def _wid():
    return lax.axis_index("s") * NC + lax.axis_index("c")


def _gather_body(off, epw, h_tab, xs_tab, srcf, dstf, ghs_out, ghd_out,
                 xp_out, idx_s, idx_d, bufs, bufd, bufxs, bufxd, xpbuf,
                 semi0, semi1, semg0, semg1, semw0, semw1):
    base0 = _wid() * epw
    nb_blocks = epw // K
    semi = (semi0, semi1)
    semg = (semg0, semg1)
    semw = (semw0, semw1)

    def idx_refs(b):
        return idx_s.at[b], idx_d.at[b]

    def issue_idx(j, b):
        base = off + base0 + j * K
        ii_s, ii_d = idx_refs(b)
        pltpu.async_copy(srcf.at[pl.ds(base, K)], ii_s, semi[b])
        pltpu.async_copy(dstf.at[pl.ds(base, K)], ii_d, semi[b])

    def wait_idx(b):
        ii_s, ii_d = idx_refs(b)
        pltpu.make_async_copy(srcf.at[pl.ds(0, K)], ii_s, semi[b]).wait()
        pltpu.make_async_copy(dstf.at[pl.ds(0, K)], ii_d, semi[b]).wait()

    def issue_gather(b):
        ii_s, ii_d = idx_refs(b)
        pltpu.async_copy(h_tab.at[ii_s], bufs.at[b], semg[b])
        pltpu.async_copy(h_tab.at[ii_d], bufd.at[b], semg[b])
        pltpu.async_copy(xs_tab.at[ii_s], bufxs.at[b], semg[b])
        pltpu.async_copy(xs_tab.at[ii_d], bufxd.at[b], semg[b])

    def wait_gather(b):
        ii_s, ii_d = idx_refs(b)
        pltpu.make_async_copy(h_tab.at[ii_s], bufs.at[b], semg[b]).wait()
        pltpu.make_async_copy(h_tab.at[ii_d], bufd.at[b], semg[b]).wait()
        pltpu.make_async_copy(xs_tab.at[ii_s], bufxs.at[b], semg[b]).wait()
        pltpu.make_async_copy(xs_tab.at[ii_d], bufxd.at[b], semg[b]).wait()

    def issue_write(j, b):
        base = base0 + j * K
        pltpu.async_copy(bufs.at[b], ghs_out.at[pl.ds(base, K)], semw[b])
        pltpu.async_copy(bufd.at[b], ghd_out.at[pl.ds(base, K)], semw[b])
        pltpu.async_copy(xpbuf.at[b], xp_out.at[pl.ds(base, K)], semw[b])

    def wait_write(b):
        pltpu.make_async_copy(bufs.at[b], ghs_out.at[pl.ds(0, K)],
                              semw[b]).wait()
        pltpu.make_async_copy(bufd.at[b], ghd_out.at[pl.ds(0, K)],
                              semw[b]).wait()
        pltpu.make_async_copy(xpbuf.at[b], xp_out.at[pl.ds(0, K)],
                              semw[b]).wait()

    def pack(b):
        def row(i, pc):
            xpbuf[b, i, pl.ds(0, STAT)] = bufxs[b, i, pl.ds(0, STAT)]
            xpbuf[b, i, pl.ds(STAT, STAT)] = bufxd[b, i, pl.ds(0, STAT)]
            return pc

        lax.fori_loop(0, K, row, 0)

    issue_idx(0, 0)
    issue_idx(1, 1)
    wait_idx(0)
    issue_gather(0)

    def body(j, carry):
        def steady(b, nb):
            wait_gather(b)
            pack(b)

            @pl.when(j >= 1)
            def _():
                wait_write(nb)

            @pl.when(j + 1 < nb_blocks)
            def _():
                wait_idx(nb)
                issue_gather(nb)

            @pl.when(j + 2 < nb_blocks)
            def _():
                issue_idx(j + 2, b)

            issue_write(j, b)

        @pl.when(lax.rem(j, 2) == 0)
        def _():
            steady(0, 1)

        @pl.when(lax.rem(j, 2) == 1)
        def _():
            steady(1, 0)

        return carry

    lax.fori_loop(0, nb_blocks, body, 0)
    wait_write((nb_blocks - 1) % 2)


def _scatter_body(off, epw, gs, dstf, expsf, zrows, part_out, idxd, gbuf,
                  ebuf, msgbuf, acc, semL0, semL1, semS):
    c = lax.axis_index("c")
    s = lax.axis_index("s")
    wid = s * NC + c
    pltpu.sync_copy(zrows, acc.at[pl.ds(s * ZCH, ZCH)])
    ebuf[0, pl.ds(K * 8, 16)] = jnp.zeros((16,), jnp.float32)
    ebuf[1, pl.ds(K * 8, 16)] = jnp.zeros((16,), jnp.float32)
    plsc.subcore_barrier()

    nb_blocks = epw // K
    semL = (semL0, semL1)

    def issue_loads(j, b):
        base = wid * epw + j * K
        pltpu.async_copy(dstf.at[pl.ds(off + base, K)], idxd.at[b], semL[b])
        pltpu.async_copy(gs.at[pl.ds(base, K)], gbuf.at[b], semL[b])
        pltpu.async_copy(expsf.at[pl.ds(base * 8, K * 8)],
                         ebuf.at[b].at[pl.ds(0, K * 8)], semL[b])

    def wait_loads(b):
        pltpu.make_async_copy(dstf.at[pl.ds(0, K)], idxd.at[b],
                              semL[b]).wait()
        pltpu.make_async_copy(gs.at[pl.ds(0, K)], gbuf.at[b], semL[b]).wait()
        pltpu.make_async_copy(expsf.at[pl.ds(0, K * 8)],
                              ebuf.at[b].at[pl.ds(0, K * 8)], semL[b]).wait()

    def wait_scatter(b):
        pltpu.make_async_copy(msgbuf, acc.at[idxd.at[b]], semS).wait()

    issue_loads(0, 0)

    def body(j, carry):
        def steady(b, nb):
            wait_loads(b)

            @pl.when(j >= 1)
            def _():
                wait_scatter(nb)

            @pl.when(j + 1 < nb_blocks)
            def _():
                issue_loads(j + 1, nb)

            def row(i, rc):
                erow = ebuf[b, pl.ds(8 * i, 16)]
                for v in range(NUM_HEADS):
                    ev = jnp.full((16,), erow[v], dtype=jnp.float32)
                    hv = gbuf[b, i, pl.ds(16 * v, 16)]
                    msgbuf[i, pl.ds(16 * v, 16)] = hv * ev
                msgbuf[i, pl.ds(HID, 16)] = erow
                return rc

            lax.fori_loop(0, K, row, 0)
            pltpu.async_copy(msgbuf, acc.at[idxd.at[b]], semS, add=True)

        @pl.when(lax.rem(j, 2) == 0)
        def _():
            steady(0, 1)

        @pl.when(lax.rem(j, 2) == 1)
        def _():
            steady(1, 0)

        return carry

    lax.fori_loop(0, nb_blocks, body, 0)
    wait_scatter((nb_blocks - 1) % 2)
    plsc.subcore_barrier()
    pltpu.sync_copy(acc.at[pl.ds(s * DCH, DCH)],
                    part_out.at[c].at[pl.ds(s * DCH, DCH)])


def _recip_gather_body(recip, dstf, out, idxd, rbuf, semi0, semi1, semg0,
                       semg1, semw0, semw1):
    base0 = _wid() * (E // NW)
    semi = (semi0, semi1)
    semg = (semg0, semg1)
    semw = (semw0, semw1)

    def issue_idx(j, b):
        pltpu.async_copy(dstf.at[pl.ds(base0 + j * KR, KR)], idxd.at[b],
                         semi[b])

    def wait_idx(b):
        pltpu.make_async_copy(dstf.at[pl.ds(0, KR)], idxd.at[b],
                              semi[b]).wait()

    def issue_gather(b):
        pltpu.async_copy(recip.at[idxd.at[b]], rbuf.at[b], semg[b])

    def wait_gather(b):
        pltpu.make_async_copy(recip.at[idxd.at[b]], rbuf.at[b],
                              semg[b]).wait()

    def issue_write(j, b):
        pltpu.async_copy(rbuf.at[b], out.at[pl.ds(base0 + j * KR, KR)],
                         semw[b])

    def wait_write(b):
        pltpu.make_async_copy(rbuf.at[b], out.at[pl.ds(0, KR)],
                              semw[b]).wait()

    issue_idx(0, 0)
    issue_idx(1, 1)
    wait_idx(0)
    issue_gather(0)

    def body(j, carry):
        def steady(b, nb):
            wait_gather(b)

            @pl.when(j >= 1)
            def _():
                wait_write(nb)

            @pl.when(j + 1 < NBR)
            def _():
                wait_idx(nb)
                issue_gather(nb)

            @pl.when(j + 2 < NBR)
            def _():
                issue_idx(j + 2, b)

            issue_write(j, b)

        @pl.when(lax.rem(j, 2) == 0)
        def _():
            steady(0, 1)

        @pl.when(lax.rem(j, 2) == 1)
        def _():
            steady(1, 0)

        return carry

    lax.fori_loop(0, NBR, body, 0)
    wait_write((NBR - 1) % 2)


def _mlp_body(gs_ref, gd_ref, xp_ref, ef_ref, w1s_ref, w1d_ref, w1x_ref,
              w1e_ref, w2_ref, out_ref):
    bf16 = jnp.bfloat16
    pre = jnp.dot(gs_ref[...].astype(bf16), w1s_ref[...],
                  preferred_element_type=jnp.float32)
    pre = pre + jnp.dot(gd_ref[...].astype(bf16), w1d_ref[...],
                        preferred_element_type=jnp.float32)
    pre = pre + jnp.dot(xp_ref[:, :2 * STAT].astype(bf16), w1x_ref[...],
                        preferred_element_type=jnp.float32)
    pre = pre + jnp.dot(ef_ref[...].astype(bf16), w1e_ref[...],
                        preferred_element_type=jnp.float32)
    z = jnp.maximum(pre, 0.0)
    raw = jnp.dot(z.astype(bf16), w2_ref[...],
                  preferred_element_type=jnp.float32)
    sc = jnp.maximum(raw, 0.01 * raw) * (1.0 / jnp.sqrt(jnp.float32(HEAD_SIZE)))
    out_ref[...] = jnp.exp(sc)


def _reduce_body(pa0_ref, pa1_ref, pb0_ref, pb1_ref, r_ref, wout_ref,
                 proj_ref, recip_ref):
    tot = (pa0_ref[...] + pa1_ref[...]) + (pb0_ref[...] + pb1_ref[...])
    u = tot[:, :HID]
    se = tot[:, HID:HID + NUM_HEADS]
    rec = 1.0 / (se + 1e-9)
    recip_ref[...] = rec
    rep = jnp.dot(rec, r_ref[...], preferred_element_type=jnp.float32)
    proj_ref[...] = jnp.dot(u * rep, wout_ref[...],
                            preferred_element_type=jnp.float32)


def _mul_body(a_ref, b_ref, o_ref):
    o_ref[...] = a_ref[...] * b_ref[...]


def kernel(h, x_s, edge_index, edge_features, W1, W2, W_out):
    f32 = jnp.float32
    bf16 = jnp.bfloat16
    src = edge_index[0]
    dst = edge_index[1]
    w1sT = W1[:, :HID].T.astype(bf16)
    w1dT = W1[:, HID:2 * HID].T.astype(bf16)
    w1xT = W1[:, 2 * HID:2 * HID + 2 * STAT].T.astype(bf16)
    w1eT = W1[:, 2 * HID + 2 * STAT:].T.astype(bf16)
    w2T = W2.T.astype(bf16)
    woutT = W_out.T
    rbc = jnp.repeat(jnp.eye(NUM_HEADS, dtype=f32), HEAD_SIZE, axis=1)
    zrows = jnp.zeros((ZCH, ROW), f32)

    mesh = plsc.VectorSubcoreMesh(**_SC_MESH)

    def gather_half(off, eh):
        return pl.kernel(
            functools.partial(_gather_body, off, eh // NW),
            out_type=(jax.ShapeDtypeStruct((eh, HID), f32),
                      jax.ShapeDtypeStruct((eh, HID), f32),
                      jax.ShapeDtypeStruct((eh, HID), f32)),
            mesh=mesh, **_CP,
            scratch_types=[
                pltpu.VMEM((2, K), jnp.int32),
                pltpu.VMEM((2, K), jnp.int32),
                pltpu.VMEM((2, K, HID), f32),
                pltpu.VMEM((2, K, HID), f32),
                pltpu.VMEM((2, K, STAT), f32),
                pltpu.VMEM((2, K, STAT), f32),
                pltpu.VMEM((2, K, HID), f32),
                pltpu.SemaphoreType.DMA,
                pltpu.SemaphoreType.DMA,
                pltpu.SemaphoreType.DMA,
                pltpu.SemaphoreType.DMA,
                pltpu.SemaphoreType.DMA,
                pltpu.SemaphoreType.DMA,
            ],
        )(h, x_s, src, dst)

    be = 2000

    def mlp_half(gs, gd, xp, ef_half):
        eh = gs.shape[0]
        return pl.pallas_call(
            _mlp_body,
            grid=(eh // be,),
            in_specs=[
                pl.BlockSpec((be, HID), lambda i: (i, 0)),
                pl.BlockSpec((be, HID), lambda i: (i, 0)),
                pl.BlockSpec((be, HID), lambda i: (i, 0)),
                pl.BlockSpec((be, EFEAT), lambda i: (i, 0)),
                pl.BlockSpec((HID, MLP_WIDTH), lambda i: (0, 0)),
                pl.BlockSpec((HID, MLP_WIDTH), lambda i: (0, 0)),
                pl.BlockSpec((2 * STAT, MLP_WIDTH), lambda i: (0, 0)),
                pl.BlockSpec((EFEAT, MLP_WIDTH), lambda i: (0, 0)),
                pl.BlockSpec((MLP_WIDTH, NUM_HEADS), lambda i: (0, 0)),
            ],
            out_specs=pl.BlockSpec((be, NUM_HEADS), lambda i: (i, 0)),
            out_shape=jax.ShapeDtypeStruct((eh, NUM_HEADS), f32),
        )(gs, gd, xp, ef_half, w1sT, w1dT, w1xT, w1eT, w2T)

    def scatter_half(off, gs, exps):
        return pl.kernel(
            functools.partial(_scatter_body, off, gs.shape[0] // NW),
            out_type=jax.ShapeDtypeStruct((NC, N, ROW), f32),
            mesh=mesh, **_CP,
            scratch_types=[
                pltpu.VMEM((2, K), jnp.int32),
                pltpu.VMEM((2, K, HID), f32),
                pltpu.VMEM((2, K * 8 + 16), f32),
                pltpu.VMEM((K, ROW), f32),
                pltpu.VMEM_SHARED((NPAD, ROW), f32),
                pltpu.SemaphoreType.DMA,
                pltpu.SemaphoreType.DMA,
                pltpu.SemaphoreType.DMA,
            ],
        )(gs, dst, exps.reshape(-1), zrows)

    gsa, gda, xpa = gather_half(0, EA)
    gsb, gdb, xpb = gather_half(EA, EB)
    expsa = mlp_half(gsa, gda, xpa, edge_features[:EA])
    expsb = mlp_half(gsb, gdb, xpb, edge_features[EA:])
    parta = scatter_half(0, gsa, expsa)
    partb = scatter_half(EA, gsb, expsb)

    bn = 400
    proj, recip = pl.pallas_call(
        _reduce_body,
        grid=(N // bn,),
        in_specs=[
            pl.BlockSpec((bn, ROW), lambda i: (i, 0)),
            pl.BlockSpec((bn, ROW), lambda i: (i, 0)),
            pl.BlockSpec((bn, ROW), lambda i: (i, 0)),
            pl.BlockSpec((bn, ROW), lambda i: (i, 0)),
            pl.BlockSpec((NUM_HEADS, HID), lambda i: (0, 0)),
            pl.BlockSpec((HID, HID), lambda i: (0, 0)),
        ],
        out_specs=[
            pl.BlockSpec((bn, HID), lambda i: (i, 0)),
            pl.BlockSpec((bn, NUM_HEADS), lambda i: (i, 0)),
        ],
        out_shape=[
            jax.ShapeDtypeStruct((N, HID), f32),
            jax.ShapeDtypeStruct((N, NUM_HEADS), f32),
        ],
    )(parta[0], parta[1], partb[0], partb[1], rbc, woutT)

    recipg = pl.kernel(
        _recip_gather_body,
        out_type=jax.ShapeDtypeStruct((E, NUM_HEADS), f32),
        mesh=mesh, **_CP,
        scratch_types=[
            pltpu.VMEM((2, KR), jnp.int32),
            pltpu.VMEM((2, KR, NUM_HEADS), f32),
            pltpu.SemaphoreType.DMA,
            pltpu.SemaphoreType.DMA,
            pltpu.SemaphoreType.DMA,
            pltpu.SemaphoreType.DMA,
            pltpu.SemaphoreType.DMA,
            pltpu.SemaphoreType.DMA,
        ],
    )(recip, dst)

    exps = jnp.concatenate([expsa, expsb], axis=0)
    rows = E * NUM_HEADS // 128
    bw = 2000
    weights = pl.pallas_call(
        _mul_body,
        grid=(rows // bw,),
        in_specs=[
            pl.BlockSpec((bw, 128), lambda i: (i, 0)),
            pl.BlockSpec((bw, 128), lambda i: (i, 0)),
        ],
        out_specs=pl.BlockSpec((bw, 128), lambda i: (i, 0)),
        out_shape=jax.ShapeDtypeStruct((rows, 128), f32),
    )(exps.reshape(rows, 128), recipg.reshape(rows, 128))

    return (proj, weights.reshape(E, NUM_HEADS))
